# baseline (speedup 1.0000x reference)
import functools

import jax
import jax.numpy as jnp
from jax import lax
from jax.experimental import pallas as pl
from jax.experimental.pallas import tpu as pltpu
from jax.experimental.pallas import tpu_sc as plsc

B = 4
S = 2048
D = 1024
NW = 32
SPW = S // NW
CHUNK = 32
STEPS = (B * SPW) // CHUNK
LANES = 16

_mesh = plsc.VectorSubcoreMesh(core_axis_name="c", subcore_axis_name="s")


@functools.partial(
    pl.kernel,
    out_type=jax.ShapeDtypeStruct((B * S, D), jnp.float32),
    mesh=_mesh,
    scratch_types=[
        pltpu.VMEM((STEPS, CHUNK), jnp.int32),
        pltpu.VMEM((SPW, D), jnp.float32),
        pltpu.VMEM((CHUNK, D), jnp.float32),
        pltpu.SemaphoreType.DMA,
    ],
)
def _emb_kernel(x_hbm, tok_hbm, pos_hbm, out_hbm, idx_v, pos_v, tok_v, sem):
    cid = lax.axis_index("c")
    sid = lax.axis_index("s")
    wid = sid * 2 + cid

    pltpu.sync_copy(x_hbm.at[wid], idx_v)
    pltpu.sync_copy(pos_hbm.at[pl.ds(wid * SPW, SPW)], pos_v)

    def step(t, _):
        b = t // 2
        half = t % 2
        pltpu.async_copy(tok_hbm.at[idx_v.at[t]], tok_v, sem).wait()
        def row(i, _):
            p = half * CHUNK + i
            for j in range(D // LANES):
                sl = pl.ds(j * LANES, LANES)
                tok_v[i, sl] = tok_v[i, sl] + pos_v[p, sl]
            return 0
        lax.fori_loop(0, CHUNK, row, 0)
        row_base = b * S + wid * SPW + half * CHUNK
        pltpu.sync_copy(tok_v, out_hbm.at[pl.ds(row_base, CHUNK)])
        return 0

    lax.fori_loop(0, STEPS, step, 0)


def kernel(x, token_table, pos_table):
    xr = (
        x.astype(jnp.int32)
        .reshape(B, NW, 2, CHUNK)
        .transpose(1, 0, 2, 3)
        .reshape(NW, STEPS, CHUNK)
    )
    out = _emb_kernel(xr, token_table, pos_table)
    return out.reshape(B, S, D)


# speedup vs baseline: 1.5164x; 1.5164x over previous
import functools

import jax
import jax.numpy as jnp
from jax import lax
from jax.experimental import pallas as pl
from jax.experimental.pallas import tpu as pltpu
from jax.experimental.pallas import tpu_sc as plsc

B = 4
S = 2048
D = 1024
NW = 32
SPW = S // NW
CHUNK = 16
PIECES = SPW // CHUNK
STEPS = B * PIECES
LANES = 16

_mesh = plsc.VectorSubcoreMesh(core_axis_name="c", subcore_axis_name="s")


@functools.partial(
    pl.kernel,
    out_type=jax.ShapeDtypeStruct((B * S, D), jnp.float32),
    mesh=_mesh,
    scratch_types=[
        pltpu.VMEM((STEPS, CHUNK), jnp.int32),
        pltpu.VMEM((SPW, D), jnp.float32),
        pltpu.VMEM((CHUNK, D), jnp.float32),
        pltpu.VMEM((CHUNK, D), jnp.float32),
        pltpu.SemaphoreType.DMA,
        pltpu.SemaphoreType.DMA,
        pltpu.SemaphoreType.DMA,
        pltpu.SemaphoreType.DMA,
        pltpu.SemaphoreType.DMA,
    ],
)
def _emb_kernel(x_hbm, tok_hbm, pos_hbm, out_hbm, idx_v, pos_v,
                buf0, buf1, g0, g1, w0, w1, psem):
    cid = lax.axis_index("c")
    sid = lax.axis_index("s")
    wid = sid * 2 + cid

    bufs = (buf0, buf1)
    gsems = (g0, g1)
    wsems = (w0, w1)

    pltpu.sync_copy(x_hbm.at[wid], idx_v)
    pos_load = pltpu.async_copy(pos_hbm.at[pl.ds(wid * SPW, SPW)], pos_v, psem)
    gd = [None, None]
    wd = [None, None]
    gd[0] = pltpu.async_copy(tok_hbm.at[idx_v.at[0]], buf0, g0)
    pos_load.wait()

    for t in range(STEPS):
        p = t % 2
        q = 1 - p
        b, piece = divmod(t, PIECES)
        if t + 1 < STEPS:
            if wd[q] is not None:
                wd[q].wait()
            gd[q] = pltpu.async_copy(tok_hbm.at[idx_v.at[t + 1]], bufs[q], gsems[q])
        gd[p].wait()
        buf = bufs[p]
        def row(i, _):
            pr = piece * CHUNK + i
            for j in range(D // LANES):
                sl = pl.ds(j * LANES, LANES)
                buf[i, sl] = buf[i, sl] + pos_v[pr, sl]
            return 0
        lax.fori_loop(0, CHUNK, row, 0)
        row_base = b * S + wid * SPW + piece * CHUNK
        wd[p] = pltpu.async_copy(buf, out_hbm.at[pl.ds(row_base, CHUNK)], wsems[p])

    wd[0].wait()
    wd[1].wait()


def kernel(x, token_table, pos_table):
    xr = (
        x.astype(jnp.int32)
        .reshape(B, NW, PIECES, CHUNK)
        .transpose(1, 0, 2, 3)
        .reshape(NW, STEPS, CHUNK)
    )
    out = _emb_kernel(xr, token_table, pos_table)
    return out.reshape(B, S, D)


# speedup vs baseline: 1.7062x; 1.1252x over previous
import functools

import jax
import jax.numpy as jnp
from jax import lax
from jax.experimental import pallas as pl
from jax.experimental.pallas import tpu as pltpu
from jax.experimental.pallas import tpu_sc as plsc

B = 4
S = 2048
D = 1024
NW = 32
SPW = S // NW
CHUNK = 16
PIECES = SPW // CHUNK
STEPS = B * PIECES
LANES = 16

_mesh = plsc.VectorSubcoreMesh(core_axis_name="c", subcore_axis_name="s")


@functools.partial(
    pl.kernel,
    out_type=jax.ShapeDtypeStruct((B * S, D), jnp.float32),
    mesh=_mesh,
    scratch_types=[
        pltpu.VMEM((B, SPW), jnp.int32),
        pltpu.VMEM((SPW, D), jnp.float32),
        pltpu.VMEM((CHUNK, D), jnp.float32),
        pltpu.VMEM((CHUNK, D), jnp.float32),
        pltpu.SemaphoreType.DMA,
        pltpu.SemaphoreType.DMA,
        pltpu.SemaphoreType.DMA,
        pltpu.SemaphoreType.DMA,
        pltpu.SemaphoreType.DMA,
    ],
)
def _emb_kernel(x_hbm, tok_hbm, pos_hbm, out_hbm, idx_v, pos_v,
                buf0, buf1, g0, g1, w0, w1, psem):
    cid = lax.axis_index("c")
    sid = lax.axis_index("s")
    wid = sid * 2 + cid

    bufs = (buf0, buf1)
    gsems = (g0, g1)
    wsems = (w0, w1)

    for b in range(B):
        pltpu.sync_copy(x_hbm.at[b, pl.ds(wid * SPW, SPW)], idx_v.at[b])
    pos_load = pltpu.async_copy(pos_hbm.at[pl.ds(wid * SPW, SPW)], pos_v, psem)

    def idx_slice(t):
        b, piece = divmod(t, PIECES)
        return idx_v.at[b, pl.ds(piece * CHUNK, CHUNK)]

    gd = [None, None]
    wd = [None, None]
    gd[0] = pltpu.async_copy(tok_hbm.at[idx_slice(0)], buf0, g0)
    pos_load.wait()

    for t in range(STEPS):
        p = t % 2
        q = 1 - p
        b, piece = divmod(t, PIECES)
        if t + 1 < STEPS:
            if wd[q] is not None:
                wd[q].wait()
            gd[q] = pltpu.async_copy(tok_hbm.at[idx_slice(t + 1)], bufs[q], gsems[q])
        gd[p].wait()
        buf = bufs[p]
        def row(i, _):
            pr = piece * CHUNK + i
            for j in range(D // LANES):
                sl = pl.ds(j * LANES, LANES)
                plsc.addupdate(buf.at[i, sl], pos_v[pr, sl])
            return 0
        lax.fori_loop(0, CHUNK, row, 0)
        row_base = b * S + wid * SPW + piece * CHUNK
        wd[p] = pltpu.async_copy(buf, out_hbm.at[pl.ds(row_base, CHUNK)], wsems[p])

    wd[0].wait()
    wd[1].wait()


def kernel(x, token_table, pos_table):
    out = _emb_kernel(x.astype(jnp.int32), token_table, pos_table)
    return out.reshape(B, S, D)


# speedup vs baseline: 2.0419x; 1.1968x over previous
import functools

import jax
import jax.numpy as jnp
from jax import lax
from jax.experimental import pallas as pl
from jax.experimental.pallas import tpu as pltpu
from jax.experimental.pallas import tpu_sc as plsc

B = 4
S = 2048
D = 1024
NW = 32
SPW = S // NW
CHUNK = 32
PIECES = SPW // CHUNK
STEPS = B * PIECES
LANES = 16

_mesh = plsc.VectorSubcoreMesh(core_axis_name="c", subcore_axis_name="s")


@functools.partial(
    pl.kernel,
    out_type=jax.ShapeDtypeStruct((B * S, D), jnp.float32),
    mesh=_mesh,
    scratch_types=[
        pltpu.VMEM((B, SPW), jnp.int32),
        pltpu.VMEM((CHUNK, D), jnp.float32),
        pltpu.VMEM((CHUNK, D), jnp.float32),
        pltpu.VMEM((CHUNK, D), jnp.float32),
        pltpu.SemaphoreType.DMA,
        pltpu.SemaphoreType.DMA,
        pltpu.SemaphoreType.DMA,
        pltpu.SemaphoreType.DMA,
        pltpu.SemaphoreType.DMA,
    ],
)
def _emb_kernel(x_hbm, tok_hbm, pos_hbm, out_hbm, idx_v, pos_v,
                buf0, buf1, g0, g1, w0, w1, psem):
    cid = lax.axis_index("c")
    sid = lax.axis_index("s")
    wid = sid * 2 + cid

    bufs = (buf0, buf1)
    gsems = (g0, g1)
    wsems = (w0, w1)

    for b in range(B):
        pltpu.sync_copy(x_hbm.at[b, pl.ds(wid * SPW, SPW)], idx_v.at[b])

    def pos_piece_load(piece):
        return pltpu.async_copy(
            pos_hbm.at[pl.ds(wid * SPW + piece * CHUNK, CHUNK)], pos_v, psem)

    def idx_slice(t):
        piece, b = divmod(t, B)
        return idx_v.at[b, pl.ds(piece * CHUNK, CHUNK)]

    pd = pos_piece_load(0)
    gd = [None, None]
    wd = [None, None]
    gd[0] = pltpu.async_copy(tok_hbm.at[idx_slice(0)], buf0, g0)
    pd.wait()

    for t in range(STEPS):
        p = t % 2
        q = 1 - p
        piece, b = divmod(t, B)
        if t + 1 < STEPS:
            if wd[q] is not None:
                wd[q].wait()
            gd[q] = pltpu.async_copy(tok_hbm.at[idx_slice(t + 1)], bufs[q], gsems[q])
        gd[p].wait()
        buf = bufs[p]

        @plsc.parallel_loop(0, CHUNK)
        def _add(i):
            for j in range(D // LANES):
                sl = pl.ds(j * LANES, LANES)
                plsc.addupdate(buf.at[i, sl], pos_v[i, sl])

        row_base = b * S + wid * SPW + piece * CHUNK
        wd[p] = pltpu.async_copy(buf, out_hbm.at[pl.ds(row_base, CHUNK)], wsems[p])
        if b == B - 1 and piece + 1 < PIECES:
            wd[p].wait()
            del wd
            wd = [None, None]
            pd = pos_piece_load(piece + 1)
            pd.wait()

    wd[0].wait()
    if wd[1] is not None:
        wd[1].wait()


def kernel(x, token_table, pos_table):
    out = _emb_kernel(x.astype(jnp.int32), token_table, pos_table)
    return out.reshape(B, S, D)
